# baseline (speedup 1.0000x reference)
import jax
import jax.numpy as jnp
from jax import lax
from jax.experimental import pallas as pl
from jax.experimental.pallas import tpu as pltpu
from jax.experimental.pallas import tpu_sc as plsc

_N = 524288
_LANES = 16

_info = plsc.get_sparse_core_info()
_NC = _info.num_cores
_NS = _info.num_subcores
_NW = _NC * _NS
_CHUNK = _N // _NW
_NBUF = 8
_SUB = _CHUNK // _NBUF


def _clamp_body(x_hbm, vals_hbm, out_hbm, vals_v, buf_v, in_sem, out_sem):
    wid = lax.axis_index("s") * _NC + lax.axis_index("c")
    base = wid * _CHUNK

    in_copies = [
        pltpu.async_copy(
            x_hbm.at[pl.ds(base + j * _SUB, _SUB)],
            buf_v.at[pl.ds(j * _SUB, _SUB)],
            in_sem,
        )
        for j in range(_NBUF)
    ]

    pltpu.sync_copy(vals_hbm, vals_v)
    head = vals_v[pl.ds(0, _LANES)]
    tail = vals_v[pl.ds(64 - _LANES, _LANES)]
    lo_vec = jnp.full((_LANES,), head[0], jnp.float32)
    hi_vec = jnp.full((_LANES,), tail[_LANES - 1], jnp.float32)

    out_copies = []
    for j in range(_NBUF):
        in_copies[j].wait()

        @plsc.parallel_loop(j * _SUB, (j + 1) * _SUB, _LANES, unroll=32)
        def _(i):
            sl = pl.ds(i, _LANES)
            buf_v[sl] = jnp.minimum(jnp.maximum(buf_v[sl], lo_vec), hi_vec)

        out_copies.append(
            pltpu.async_copy(
                buf_v.at[pl.ds(j * _SUB, _SUB)],
                out_hbm.at[pl.ds(base + j * _SUB, _SUB)],
                out_sem,
            )
        )

    for c in out_copies:
        c.wait()


@jax.jit
def kernel(tensor_input, values):
    x = tensor_input.reshape(_N)
    out = pl.kernel(
        _clamp_body,
        out_type=jax.ShapeDtypeStruct((_N,), jnp.float32),
        mesh=plsc.VectorSubcoreMesh(core_axis_name="c", subcore_axis_name="s"),
        scratch_types=[
            pltpu.VMEM((64,), jnp.float32),
            pltpu.VMEM((_CHUNK,), jnp.float32),
            pltpu.SemaphoreType.DMA,
            pltpu.SemaphoreType.DMA,
        ],
    )(x, values)
    return out.reshape(_N, 1)


# speedup vs baseline: 1.0619x; 1.0619x over previous
import jax
import jax.numpy as jnp
from jax import lax
from jax.experimental import pallas as pl
from jax.experimental.pallas import tpu as pltpu
from jax.experimental.pallas import tpu_sc as plsc

_N = 524288
_LANES = 16

_info = plsc.get_sparse_core_info()
_NC = _info.num_cores
_NS = _info.num_subcores
_NW = _NC * _NS
_CHUNK = _N // _NW
_NBUF = 4
_SUB = _CHUNK // _NBUF


def _clamp_body(x_hbm, vals_hbm, out_hbm, vals_v, buf_v, in_sem, out_sem):
    wid = lax.axis_index("s") * _NC + lax.axis_index("c")
    base = wid * _CHUNK

    in_copies = [
        pltpu.async_copy(
            x_hbm.at[pl.ds(base + j * _SUB, _SUB)],
            buf_v.at[pl.ds(j * _SUB, _SUB)],
            in_sem,
        )
        for j in range(_NBUF)
    ]

    pltpu.sync_copy(vals_hbm, vals_v)
    head = vals_v[pl.ds(0, _LANES)]
    tail = vals_v[pl.ds(64 - _LANES, _LANES)]
    lo_vec = jnp.full((_LANES,), head[0], jnp.float32)
    hi_vec = jnp.full((_LANES,), tail[_LANES - 1], jnp.float32)

    out_copies = []
    for j in range(_NBUF):
        in_copies[j].wait()

        @plsc.parallel_loop(j * _SUB, (j + 1) * _SUB, _LANES, unroll=8)
        def _(i):
            sl = pl.ds(i, _LANES)
            buf_v[sl] = jnp.minimum(jnp.maximum(buf_v[sl], lo_vec), hi_vec)

        out_copies.append(
            pltpu.async_copy(
                buf_v.at[pl.ds(j * _SUB, _SUB)],
                out_hbm.at[pl.ds(base + j * _SUB, _SUB)],
                out_sem,
            )
        )

    for c in out_copies:
        c.wait()


@jax.jit
def kernel(tensor_input, values):
    x = tensor_input.reshape(_N)
    out = pl.kernel(
        _clamp_body,
        out_type=jax.ShapeDtypeStruct((_N,), jnp.float32),
        mesh=plsc.VectorSubcoreMesh(core_axis_name="c", subcore_axis_name="s"),
        scratch_types=[
            pltpu.VMEM((64,), jnp.float32),
            pltpu.VMEM((_CHUNK,), jnp.float32),
            pltpu.SemaphoreType.DMA,
            pltpu.SemaphoreType.DMA,
        ],
    )(x, values)
    return out.reshape(_N, 1)


# speedup vs baseline: 1.0725x; 1.0100x over previous
import jax
import jax.numpy as jnp
from jax import lax
from jax.experimental import pallas as pl
from jax.experimental.pallas import tpu as pltpu
from jax.experimental.pallas import tpu_sc as plsc

_N = 524288
_LANES = 16

_info = plsc.get_sparse_core_info()
_NC = _info.num_cores
_NS = _info.num_subcores
_NW = _NC * _NS
_CHUNK = _N // _NW
_NBUF = 2
_SUB = _CHUNK // _NBUF


def _clamp_body(x_hbm, vals_hbm, out_hbm, vals_v, buf_v, in_sem, out_sem):
    wid = lax.axis_index("s") * _NC + lax.axis_index("c")
    base = wid * _CHUNK

    in_copies = [
        pltpu.async_copy(
            x_hbm.at[pl.ds(base + j * _SUB, _SUB)],
            buf_v.at[pl.ds(j * _SUB, _SUB)],
            in_sem,
        )
        for j in range(_NBUF)
    ]

    pltpu.sync_copy(vals_hbm, vals_v)
    head = vals_v[pl.ds(0, _LANES)]
    tail = vals_v[pl.ds(64 - _LANES, _LANES)]
    lo_vec = jnp.full((_LANES,), head[0], jnp.float32)
    hi_vec = jnp.full((_LANES,), tail[_LANES - 1], jnp.float32)

    out_copies = []
    for j in range(_NBUF):
        in_copies[j].wait()

        @plsc.parallel_loop(j * _SUB, (j + 1) * _SUB, _LANES, unroll=8)
        def _(i):
            sl = pl.ds(i, _LANES)
            buf_v[sl] = jnp.minimum(jnp.maximum(buf_v[sl], lo_vec), hi_vec)

        out_copies.append(
            pltpu.async_copy(
                buf_v.at[pl.ds(j * _SUB, _SUB)],
                out_hbm.at[pl.ds(base + j * _SUB, _SUB)],
                out_sem,
            )
        )

    for c in out_copies:
        c.wait()


@jax.jit
def kernel(tensor_input, values):
    x = tensor_input.reshape(_N)
    out = pl.kernel(
        _clamp_body,
        out_type=jax.ShapeDtypeStruct((_N,), jnp.float32),
        mesh=plsc.VectorSubcoreMesh(core_axis_name="c", subcore_axis_name="s"),
        scratch_types=[
            pltpu.VMEM((64,), jnp.float32),
            pltpu.VMEM((_CHUNK,), jnp.float32),
            pltpu.SemaphoreType.DMA,
            pltpu.SemaphoreType.DMA,
        ],
    )(x, values)
    return out.reshape(_N, 1)
